# baseline (speedup 1.0000x reference)
import functools
import math

import jax
import jax.numpy as jnp
import numpy as np
from jax import lax
from jax.experimental import pallas as pl
from jax.experimental.pallas import tpu as pltpu
from jax.experimental.pallas import tpu_sc as plsc


def _sinusoidal_pe(maxlen, dim):
    pos = jnp.arange(maxlen, dtype=jnp.float32)
    i = np.arange(dim)
    terms = jnp.asarray(1.0 / (10000.0 ** (2 * (i // 2) / dim)), dtype=jnp.float32)
    pe_val = pos[:, None] * terms[None, :]
    even = pe_val[:, 0::2]
    pe = jnp.zeros((maxlen, dim), dtype=jnp.float32)
    pe = pe.at[:, 0::2].set(jnp.sin(even))
    pe = pe.at[:, 1::2].set(jnp.cos(even))
    return pe


@functools.partial(jax.jit, static_argnames=("batch", "seq", "dim", "scale"))
def _sc_gather_pe(table2, idxp, sel, pe, *, batch, seq, dim, scale):
    NW = 32
    CHUNK = seq // 2
    seqs_per_w = batch // NW
    n_rows = batch * seq
    groups = dim // 16

    mesh = plsc.VectorSubcoreMesh(core_axis_name="c", subcore_axis_name="s")

    @functools.partial(
        pl.kernel,
        mesh=mesh,
        compiler_params=pltpu.CompilerParams(use_tc_tiling_on_sc=True),
        out_type=jax.ShapeDtypeStruct((n_rows, dim), jnp.float32),
        scratch_types=[
            pltpu.VMEM((2 * seqs_per_w, CHUNK), jnp.int32),
            pltpu.VMEM((seq, 2 * dim), jnp.float32),
            pltpu.VMEM((seq, 2 * dim), jnp.float32),
            pltpu.VMEM((seq, dim), jnp.float32),
            pltpu.VMEM((seq, dim), jnp.float32),
            pltpu.VMEM((seqs_per_w, 16 * ((seq + 15) // 16)), jnp.int32),
            pltpu.SemaphoreType.DMA,
            pltpu.SemaphoreType.DMA,
        ],
    )
    def k(tab_hbm, idxp_hbm, sel_hbm, pe_hbm, out_hbm,
          idx_v, bufa, bufb, outv, pe_v, sel_v, gsem, wsem):
        wid = lax.axis_index("s") * 2 + lax.axis_index("c")
        g0 = wid * seqs_per_w
        pltpu.sync_copy(idxp_hbm.at[pl.ds(wid * 2 * seqs_per_w, 2 * seqs_per_w)],
                        idx_v)
        pltpu.sync_copy(sel_hbm.at[pl.ds(g0, seqs_per_w)], sel_v)
        pltpu.sync_copy(pe_hbm, pe_v)

        def issue_gather(k_local, buf):
            pltpu.async_copy(
                tab_hbm.at[idx_v.at[2 * k_local]], buf.at[pl.ds(0, CHUNK)], gsem
            )
            pltpu.async_copy(
                tab_hbm.at[idx_v.at[2 * k_local + 1]],
                buf.at[pl.ds(CHUNK, CHUNK)], gsem,
            )

        def drain(sem, dst_ref, dummy_src):
            pltpu.make_async_copy(dummy_src, dst_ref, sem).wait()

        def compute(k_local, buf):
            def do_rows(rbase, lanes):
                selv = sel_v[k_local, pl.ds(rbase, 16)]
                for i in range(lanes):
                    r = rbase + i
                    base = pl.multiple_of(selv[i], 16)
                    for q in range(groups):
                        outv[r, pl.ds(q * 16, 16)] = (
                            buf[r, pl.ds(base + q * 16, 16)] * scale
                            + pe_v[r, pl.ds(q * 16, 16)]
                        )

            def block_body(b, carry):
                do_rows(pl.multiple_of(b * 16, 16), 16)
                return carry

            lax.fori_loop(0, seq // 16, block_body, 0)
            if seq % 16:
                do_rows((seq // 16) * 16, seq % 16)

        def handle(k_local, buf):
            g = g0 + k_local
            drain(gsem, buf, tab_hbm.at[pl.ds(0, seq)])
            compute(k_local, buf)
            pltpu.sync_copy(outv, out_hbm.at[pl.ds(g * seq, seq)])

        issue_gather(0, bufa)

        def pair_body(k2, carry):
            ka = 2 * k2
            issue_gather(ka + 1, bufb)
            handle(ka, bufa)

            @pl.when(k2 <= (seqs_per_w // 2 - 2))
            def _():
                issue_gather(ka + 2, bufa)

            handle(ka + 1, bufb)
            return carry

        lax.fori_loop(0, seqs_per_w // 2, pair_body, 0)

    return k(table2, idxp, sel, pe)


def kernel(inp, table):
    B, S = inp.shape
    V, D = table.shape
    inp32 = inp.astype(jnp.int32)
    table2 = table.reshape(V // 2, 2 * D)
    idxp = (inp32 // 2).reshape(B * S // (S // 2), S // 2)
    sel = (inp32 % 2) * D
    pad = (-S) % 16
    if pad:
        sel = jnp.pad(sel, ((0, 0), (0, pad)))
    pe = _sinusoidal_pe(S, D)
    scale = 1.0 / math.sqrt(float(B))
    out = _sc_gather_pe(table2, idxp, sel, pe,
                        batch=B, seq=S, dim=D, scale=scale)
    return out.reshape(B, S, D)
